# baseline (speedup 1.0000x reference)
import functools

import jax
import jax.numpy as jnp
from jax import lax
from jax.experimental import pallas as pl
from jax.experimental.pallas import tpu as pltpu
from jax.experimental.pallas import tpu_sc as plsc

N = 10000
E = 320000
D = 128
H = 64

NC = 2
NS = 16
NW = NC * NS
L = 16

NPAD = 10240
RPS = NPAD // NS

KB = 128
EP = 327680
EPW = EP // NW
NB = EPW // KB
NPAIR = NB // 2

_mesh = plsc.VectorSubcoreMesh(core_axis_name="c", subcore_axis_name="s")
_params = pltpu.CompilerParams(needs_layout_passes=False)


@functools.partial(
    pl.kernel,
    out_type=(jax.ShapeDtypeStruct((NC, NPAD, D), jnp.float32),
              jax.ShapeDtypeStruct((NW * NPAD,), jnp.float32)),
    mesh=_mesh,
    scratch_types=[
        pltpu.VMEM((NB, KB), jnp.int32),
        pltpu.VMEM((NB, KB), jnp.int32),
        pltpu.VMEM((KB, D), jnp.float32),
        pltpu.VMEM((KB, D), jnp.float32),
        pltpu.VMEM((NPAD,), jnp.float32),
        pltpu.VMEM_SHARED((NPAD, D), jnp.float32),
        pltpu.SemaphoreType.DMA,
        pltpu.SemaphoreType.DMA,
    ],
    compiler_params=_params,
)
def _seg_sum(x_hbm, src_hbm, dst_hbm, agg_hbm, deg_hbm,
             sidx_v, didx_v, rows0_v, rows1_v, hist_v, agg_sh, sem0, sem1):
    cid = lax.axis_index("c")
    sid = lax.axis_index("s")
    wid = sid * NC + cid

    zero = jnp.zeros((L,), jnp.float32)

    pltpu.sync_copy(src_hbm.at[wid], sidx_v)
    pltpu.sync_copy(dst_hbm.at[wid], didx_v)

    def _zrow(r, _):
        for j in range(D // L):
            rows0_v[r, pl.ds(j * L, L)] = zero
        return ()

    lax.fori_loop(0, KB, _zrow, ())

    def _zhist(g, _):
        hist_v[pl.ds(g * L, L)] = zero
        return ()

    lax.fori_loop(0, NPAD // L, _zhist, ())

    for c in range(RPS // KB):
        pltpu.sync_copy(rows0_v, agg_sh.at[pl.ds(sid * RPS + c * KB, KB), :])

    plsc.subcore_barrier()

    def _deg(b):
        for g in range(KB // L):
            dv = didx_v[b, pl.ds(g * L, L)]
            cnt, last = plsc.scan_count(dv)
            plsc.addupdate_scatter(hist_v, [dv], cnt.astype(jnp.float32),
                                   mask=last)

    def _batch(b, _):
        pltpu.async_copy(x_hbm.at[sidx_v.at[b]], rows0_v, sem0).wait()
        _deg(b)
        pltpu.sync_copy(rows0_v, agg_sh.at[didx_v.at[b]], add=True)
        return ()

    lax.fori_loop(0, NB, _batch, ())
    plsc.subcore_barrier()

    pltpu.sync_copy(agg_sh.at[pl.ds(sid * RPS, RPS), :],
                    agg_hbm.at[cid, pl.ds(sid * RPS, RPS), :])
    pltpu.sync_copy(hist_v, deg_hbm.at[pl.ds(wid * NPAD, NPAD)])


def _node_body(agg_ref, deg_ref, wenc_ref, benc_ref, w1a_ref, w1b_ref, b1_ref,
               c_ref):
    deg = jnp.maximum(jnp.sum(deg_ref[...], axis=0), 1.0)
    s = agg_ref[0] + agg_ref[1]
    xb = s / deg
    ne = jnp.dot(xb, wenc_ref[...], preferred_element_type=jnp.float32)
    ne = jnp.maximum(ne + benc_ref[...][None, :], 0.0)
    a = jnp.dot(ne, w1a_ref[...], preferred_element_type=jnp.float32)
    b = (jnp.dot(ne, w1b_ref[...], preferred_element_type=jnp.float32)
         + b1_ref[...][None, :])
    c_ref[...] = jnp.concatenate([a, b], axis=1)


_RB = 1024


def _node_stage(agg2, deg2, W_enc, b_enc, W1a, W1b, b1):
    grid = NPAD // _RB
    return pl.pallas_call(
        _node_body,
        grid=(grid,),
        in_specs=[
            pl.BlockSpec((NC, _RB, D), lambda i: (0, i, 0)),
            pl.BlockSpec((NW, _RB, 1), lambda i: (0, i, 0)),
            pl.BlockSpec((D, D), lambda i: (0, 0)),
            pl.BlockSpec((D,), lambda i: (0,)),
            pl.BlockSpec((D, H), lambda i: (0, 0)),
            pl.BlockSpec((D, H), lambda i: (0, 0)),
            pl.BlockSpec((H,), lambda i: (0,)),
        ],
        out_specs=pl.BlockSpec((_RB, D), lambda i: (i, 0)),
        out_shape=jax.ShapeDtypeStruct((NPAD, D), jnp.float32),
    )(agg2, deg2, W_enc, b_enc, W1a, W1b, b1)


@functools.partial(
    pl.kernel,
    out_type=jax.ShapeDtypeStruct((EP,), jnp.float32),
    mesh=_mesh,
    scratch_types=[
        pltpu.VMEM((NB, KB), jnp.int32),
        pltpu.VMEM((NB, KB), jnp.int32),
        pltpu.VMEM((KB, D), jnp.float32),
        pltpu.VMEM((KB, D), jnp.float32),
        pltpu.VMEM((KB, D), jnp.float32),
        pltpu.VMEM((KB, D), jnp.float32),
        pltpu.VMEM((KB,), jnp.float32),
        pltpu.VMEM((H,), jnp.float32),
        pltpu.VMEM((L,), jnp.float32),
        pltpu.SemaphoreType.DMA,
        pltpu.SemaphoreType.DMA,
    ],
    compiler_params=_params,
)
def _edge_mlp(c_hbm, src_hbm, dst_hbm, w2_hbm, b2v_hbm, out_hbm,
              sidx_v, didx_v, cs0_v, cd0_v, cs1_v, cd1_v, out_v, w2_v, b2v_v,
              sem0, sem1):
    cid = lax.axis_index("c")
    sid = lax.axis_index("s")
    wid = sid * NC + cid

    pltpu.sync_copy(src_hbm.at[wid], sidx_v)
    pltpu.sync_copy(dst_hbm.at[wid], didx_v)
    pltpu.sync_copy(w2_hbm, w2_v)
    pltpu.sync_copy(b2v_hbm, b2v_v)
    w2 = [w2_v[pl.ds(j * L, L)] for j in range(H // L)]
    b2s = b2v_v[...]
    riota = jnp.arange(L, dtype=jnp.int32)

    def _issue(b, cs, cd, sem):
        _ = pltpu.async_copy(c_hbm.at[sidx_v.at[b]], cs, sem)
        _ = pltpu.async_copy(c_hbm.at[didx_v.at[b]], cd, sem)

    def _wait(b, cs, cd, sem):
        pltpu.make_async_copy(c_hbm.at[sidx_v.at[b]], cs, sem).wait()
        pltpu.make_async_copy(c_hbm.at[didx_v.at[b]], cd, sem).wait()

    def _compute(b, cs, cd):
        def _group(g, _):
            rows = riota + g * L
            acc = b2s
            for j in range(H // L):
                for kk in range(L):
                    k = j * L + kk
                    av = plsc.load_gather(
                        cs, [rows, jnp.full((L,), k, jnp.int32)])
                    bv = plsc.load_gather(
                        cd, [rows, jnp.full((L,), H + k, jnp.int32)])
                    w2k = jnp.full((L,), w2[j][kk], jnp.float32)
                    acc = acc + jnp.maximum(av + bv, 0.0) * w2k
            out_v[pl.ds(g * L, L)] = acc
            return ()

        lax.fori_loop(0, KB // L, _group, ())
        pltpu.sync_copy(out_v, out_hbm.at[pl.ds(wid * EPW + b * KB, KB)])

    _issue(0, cs0_v, cd0_v, sem0)

    def _pair(p, _):
        b0 = 2 * p
        b1 = b0 + 1
        _wait(b0, cs0_v, cd0_v, sem0)
        _issue(b1, cs1_v, cd1_v, sem1)
        _compute(b0, cs0_v, cd0_v)
        _wait(b1, cs1_v, cd1_v, sem1)

        @pl.when(b0 + 2 < NB)
        def _():
            _issue(b0 + 2, cs0_v, cd0_v, sem0)

        _compute(b1, cs1_v, cd1_v)
        return ()

    lax.fori_loop(0, NPAIR, _pair, ())


def kernel(x, edge_index, W_enc, b_enc, W1, b1, W2, b2):
    src = edge_index[0]
    dst = edge_index[1]
    pad = EP - E
    srcp = jnp.concatenate([src, jnp.zeros((pad,), jnp.int32)])
    dstp = jnp.concatenate([dst, jnp.full((pad,), NPAD - 1, jnp.int32)])
    src3 = srcp.reshape(NW, NB, KB)
    dst3 = dstp.reshape(NW, NB, KB)
    agg2, deg_flat = _seg_sum(x, src3, dst3)
    deg2 = deg_flat.reshape(NW, NPAD, 1)
    C = _node_stage(agg2, deg2, W_enc, b_enc, W1[:D], W1[D:], b1)
    b2v = jnp.full((L,), b2[0], jnp.float32)
    logits = _edge_mlp(C, src3, dst3, W2.reshape(H), b2v)
    return logits[:E].reshape(E, 1)


# speedup vs baseline: 2.5459x; 2.5459x over previous
import functools

import jax
import jax.numpy as jnp
from jax import lax
from jax.experimental import pallas as pl
from jax.experimental.pallas import tpu as pltpu
from jax.experimental.pallas import tpu_sc as plsc

N = 10000
E = 320000
D = 128
H = 64

NC = 2
NS = 16
NW = NC * NS
L = 16

NPAD = 10240
RPS = NPAD // NS

KB = 128
EP = 327680
EPW = EP // NW
NB = EPW // KB
NPAIR = NB // 2

_mesh = plsc.VectorSubcoreMesh(core_axis_name="c", subcore_axis_name="s")
_params = pltpu.CompilerParams(needs_layout_passes=False)


@functools.partial(
    pl.kernel,
    out_type=(jax.ShapeDtypeStruct((NC, NPAD, D), jnp.float32),
              jax.ShapeDtypeStruct((NW * NPAD,), jnp.float32)),
    mesh=_mesh,
    scratch_types=[
        pltpu.VMEM((NB, KB), jnp.int32),
        pltpu.VMEM((NB, KB), jnp.int32),
        pltpu.VMEM((KB, D), jnp.float32),
        pltpu.VMEM((KB, D), jnp.float32),
        pltpu.VMEM((NPAD,), jnp.float32),
        pltpu.VMEM_SHARED((NPAD, D), jnp.float32),
        pltpu.SemaphoreType.DMA,
        pltpu.SemaphoreType.DMA,
    ],
    compiler_params=_params,
)
def _seg_sum(x_hbm, src_hbm, dst_hbm, agg_hbm, deg_hbm,
             sidx_v, didx_v, rows0_v, rows1_v, hist_v, agg_sh, sem0, sem1):
    cid = lax.axis_index("c")
    sid = lax.axis_index("s")
    wid = sid * NC + cid

    zero = jnp.zeros((L,), jnp.float32)

    pltpu.sync_copy(src_hbm.at[wid], sidx_v)
    pltpu.sync_copy(dst_hbm.at[wid], didx_v)

    def _zrow(r, _):
        for j in range(D // L):
            rows0_v[r, pl.ds(j * L, L)] = zero
        return ()

    lax.fori_loop(0, KB, _zrow, ())

    def _zhist(g, _):
        hist_v[pl.ds(g * L, L)] = zero
        return ()

    lax.fori_loop(0, NPAD // L, _zhist, ())

    for c in range(RPS // KB):
        pltpu.sync_copy(rows0_v, agg_sh.at[pl.ds(sid * RPS + c * KB, KB), :])

    plsc.subcore_barrier()

    def _deg(b):
        for g in range(KB // L):
            dv = didx_v[b, pl.ds(g * L, L)]
            cnt, last = plsc.scan_count(dv)
            plsc.addupdate_scatter(hist_v, [dv], cnt.astype(jnp.float32),
                                   mask=last)

    def _batch(b, _):
        pltpu.async_copy(x_hbm.at[sidx_v.at[b]], rows0_v, sem0).wait()
        _deg(b)
        pltpu.sync_copy(rows0_v, agg_sh.at[didx_v.at[b]], add=True)
        return ()

    lax.fori_loop(0, NB, _batch, ())
    plsc.subcore_barrier()

    pltpu.sync_copy(agg_sh.at[pl.ds(sid * RPS, RPS), :],
                    agg_hbm.at[cid, pl.ds(sid * RPS, RPS), :])
    pltpu.sync_copy(hist_v, deg_hbm.at[pl.ds(wid * NPAD, NPAD)])


def _node_body(agg_ref, deg_ref, wenc_ref, benc_ref, w1a_ref, w1b_ref, b1_ref,
               c_ref):
    deg = jnp.maximum(jnp.sum(deg_ref[...], axis=0), 1.0)
    s = agg_ref[0] + agg_ref[1]
    xb = s / deg
    ne = jnp.dot(xb, wenc_ref[...], preferred_element_type=jnp.float32)
    ne = jnp.maximum(ne + benc_ref[...][None, :], 0.0)
    a = jnp.dot(ne, w1a_ref[...], preferred_element_type=jnp.float32)
    b = (jnp.dot(ne, w1b_ref[...], preferred_element_type=jnp.float32)
         + b1_ref[...][None, :])
    c_ref[...] = jnp.concatenate([a, b], axis=1)


_RB = 1024


def _node_stage(agg2, deg2, W_enc, b_enc, W1a, W1b, b1):
    grid = NPAD // _RB
    return pl.pallas_call(
        _node_body,
        grid=(grid,),
        in_specs=[
            pl.BlockSpec((NC, _RB, D), lambda i: (0, i, 0)),
            pl.BlockSpec((NW, _RB, 1), lambda i: (0, i, 0)),
            pl.BlockSpec((D, D), lambda i: (0, 0)),
            pl.BlockSpec((D,), lambda i: (0,)),
            pl.BlockSpec((D, H), lambda i: (0, 0)),
            pl.BlockSpec((D, H), lambda i: (0, 0)),
            pl.BlockSpec((H,), lambda i: (0,)),
        ],
        out_specs=pl.BlockSpec((_RB, D), lambda i: (i, 0)),
        out_shape=jax.ShapeDtypeStruct((NPAD, D), jnp.float32),
    )(agg2, deg2, W_enc, b_enc, W1a, W1b, b1)


@functools.partial(
    pl.kernel,
    out_type=jax.ShapeDtypeStruct((EP,), jnp.float32),
    mesh=_mesh,
    scratch_types=[
        pltpu.VMEM((NB, KB), jnp.int32),
        pltpu.VMEM((NB, KB), jnp.int32),
        pltpu.VMEM((KB, D), jnp.float32),
        pltpu.VMEM((KB, D), jnp.float32),
        pltpu.VMEM((KB, D), jnp.float32),
        pltpu.VMEM((KB, D), jnp.float32),
        pltpu.VMEM((KB,), jnp.float32),
        pltpu.VMEM((H,), jnp.float32),
        pltpu.VMEM((L,), jnp.float32),
        pltpu.SemaphoreType.DMA,
        pltpu.SemaphoreType.DMA,
    ],
    compiler_params=_params,
)
def _edge_mlp(c_hbm, src_hbm, dst_hbm, w2_hbm, b2v_hbm, out_hbm,
              sidx_v, didx_v, cs0_v, cd0_v, cs1_v, cd1_v, out_v, w2_v, b2v_v,
              sem0, sem1):
    cid = lax.axis_index("c")
    sid = lax.axis_index("s")
    wid = sid * NC + cid

    pltpu.sync_copy(src_hbm.at[wid], sidx_v)
    pltpu.sync_copy(dst_hbm.at[wid], didx_v)
    pltpu.sync_copy(w2_hbm, w2_v)
    pltpu.sync_copy(b2v_hbm, b2v_v)
    w2 = [w2_v[pl.ds(j * L, L)] for j in range(H // L)]
    b2s = b2v_v[...]
    riota = jnp.arange(L, dtype=jnp.int32)

    def _issue(b, cs, cd, sem):
        _ = pltpu.async_copy(c_hbm.at[sidx_v.at[b]], cs, sem)
        _ = pltpu.async_copy(c_hbm.at[didx_v.at[b]], cd, sem)

    def _wait(b, cs, cd, sem):
        pltpu.make_async_copy(c_hbm.at[sidx_v.at[b]], cs, sem).wait()
        pltpu.make_async_copy(c_hbm.at[didx_v.at[b]], cd, sem).wait()

    lane0 = riota == 0

    def _compute(b, cs, cd):
        def _group(g, _):
            base = g * L
            for i in range(L):
                row = base + i
                acc = b2s
                for j in range(H // L):
                    a = cs[row, pl.ds(j * L, L)]
                    bb = cd[row, pl.ds(H + j * L, L)]
                    acc = acc + jnp.maximum(a + bb, 0.0) * w2[j]
                r = jnp.sum(acc)
                plsc.store_scatter(out_v, [jnp.full((L,), row, jnp.int32)],
                                   jnp.full((L,), r, jnp.float32), mask=lane0)
            return ()

        lax.fori_loop(0, KB // L, _group, ())
        pltpu.sync_copy(out_v, out_hbm.at[pl.ds(wid * EPW + b * KB, KB)])

    _issue(0, cs0_v, cd0_v, sem0)

    def _pair(p, _):
        b0 = 2 * p
        b1 = b0 + 1
        _wait(b0, cs0_v, cd0_v, sem0)
        _issue(b1, cs1_v, cd1_v, sem1)
        _compute(b0, cs0_v, cd0_v)
        _wait(b1, cs1_v, cd1_v, sem1)

        @pl.when(b0 + 2 < NB)
        def _():
            _issue(b0 + 2, cs0_v, cd0_v, sem0)

        _compute(b1, cs1_v, cd1_v)
        return ()

    lax.fori_loop(0, NPAIR, _pair, ())


def kernel(x, edge_index, W_enc, b_enc, W1, b1, W2, b2):
    src = edge_index[0]
    dst = edge_index[1]
    pad = EP - E
    ar = jnp.arange(pad, dtype=jnp.int32)
    srcp = jnp.concatenate([src, ar % N])
    dstp = jnp.concatenate([dst, N + ar % (NPAD - N)])
    src3 = srcp.reshape(NW, NB, KB)
    dst3 = dstp.reshape(NW, NB, KB)
    agg2, deg_flat = _seg_sum(x, src3, dst3)
    deg2 = deg_flat.reshape(NW, NPAD, 1)
    C = _node_stage(agg2, deg2, W_enc, b_enc, W1[:D], W1[D:], b1)
    b2v = jnp.full((L,), b2[0], jnp.float32)
    logits = _edge_mlp(C, src3, dst3, W2.reshape(H), b2v)
    return logits[:E].reshape(E, 1)


# speedup vs baseline: 2.5838x; 1.0149x over previous
import functools

import jax
import jax.numpy as jnp
from jax import lax
from jax.experimental import pallas as pl
from jax.experimental.pallas import tpu as pltpu
from jax.experimental.pallas import tpu_sc as plsc

N = 10000
E = 320000
D = 128
H = 64

NC = 2
NS = 16
NW = NC * NS
L = 16

NPAD = 10240
RPS = NPAD // NS

KB = 128
EP = 327680
EPW = EP // NW
NB = EPW // KB
NPAIR = NB // 2

_mesh = plsc.VectorSubcoreMesh(core_axis_name="c", subcore_axis_name="s")
_params = pltpu.CompilerParams(needs_layout_passes=False)


@functools.partial(
    pl.kernel,
    out_type=(jax.ShapeDtypeStruct((NC, NPAD, D), jnp.float32),
              jax.ShapeDtypeStruct((NW * NPAD,), jnp.float32)),
    mesh=_mesh,
    scratch_types=[
        pltpu.VMEM((NB, KB), jnp.int32),
        pltpu.VMEM((NB, KB), jnp.int32),
        pltpu.VMEM((KB, D), jnp.float32),
        pltpu.VMEM((KB, D), jnp.float32),
        pltpu.VMEM((NPAD,), jnp.float32),
        pltpu.VMEM_SHARED((NPAD, D), jnp.float32),
        pltpu.SemaphoreType.DMA,
        pltpu.SemaphoreType.DMA,
    ],
    compiler_params=_params,
)
def _seg_sum(x_hbm, src_hbm, dst_hbm, agg_hbm, deg_hbm,
             sidx_v, didx_v, rows0_v, rows1_v, hist_v, agg_sh, sem0, sem1):
    cid = lax.axis_index("c")
    sid = lax.axis_index("s")
    wid = sid * NC + cid

    zero = jnp.zeros((L,), jnp.float32)

    pltpu.sync_copy(src_hbm.at[wid], sidx_v)
    pltpu.sync_copy(dst_hbm.at[wid], didx_v)

    def _zrow(r, _):
        for j in range(D // L):
            rows0_v[r, pl.ds(j * L, L)] = zero
        return ()

    lax.fori_loop(0, KB, _zrow, ())

    def _zhist(g, _):
        hist_v[pl.ds(g * L, L)] = zero
        return ()

    lax.fori_loop(0, NPAD // L, _zhist, ())

    for c in range(RPS // KB):
        pltpu.sync_copy(rows0_v, agg_sh.at[pl.ds(sid * RPS + c * KB, KB), :])

    plsc.subcore_barrier()

    def _deg(b):
        for g in range(KB // L):
            dv = didx_v[b, pl.ds(g * L, L)]
            cnt, last = plsc.scan_count(dv)
            plsc.addupdate_scatter(hist_v, [dv], cnt.astype(jnp.float32),
                                   mask=last)

    def _batch(b, _):
        d = pltpu.async_copy(x_hbm.at[sidx_v.at[b]], rows0_v, sem0)
        _deg(b)
        d.wait()
        pltpu.sync_copy(rows0_v, agg_sh.at[didx_v.at[b]], add=True)
        return ()

    lax.fori_loop(0, NB, _batch, ())
    plsc.subcore_barrier()

    pltpu.sync_copy(agg_sh.at[pl.ds(sid * RPS, RPS), :],
                    agg_hbm.at[cid, pl.ds(sid * RPS, RPS), :])
    pltpu.sync_copy(hist_v, deg_hbm.at[pl.ds(wid * NPAD, NPAD)])


def _node_body(agg_ref, deg_ref, wenc_ref, benc_ref, w1a_ref, w1b_ref, b1_ref,
               c_ref):
    deg = jnp.maximum(jnp.sum(deg_ref[...], axis=0), 1.0)
    s = agg_ref[0] + agg_ref[1]
    xb = s / deg
    ne = jnp.dot(xb, wenc_ref[...], preferred_element_type=jnp.float32)
    ne = jnp.maximum(ne + benc_ref[...][None, :], 0.0)
    a = jnp.dot(ne, w1a_ref[...], preferred_element_type=jnp.float32)
    b = (jnp.dot(ne, w1b_ref[...], preferred_element_type=jnp.float32)
         + b1_ref[...][None, :])
    c_ref[...] = jnp.concatenate([a, b], axis=1)


_RB = 1024


def _node_stage(agg2, deg2, W_enc, b_enc, W1a, W1b, b1):
    grid = NPAD // _RB
    return pl.pallas_call(
        _node_body,
        grid=(grid,),
        in_specs=[
            pl.BlockSpec((NC, _RB, D), lambda i: (0, i, 0)),
            pl.BlockSpec((NW, _RB, 1), lambda i: (0, i, 0)),
            pl.BlockSpec((D, D), lambda i: (0, 0)),
            pl.BlockSpec((D,), lambda i: (0,)),
            pl.BlockSpec((D, H), lambda i: (0, 0)),
            pl.BlockSpec((D, H), lambda i: (0, 0)),
            pl.BlockSpec((H,), lambda i: (0,)),
        ],
        out_specs=pl.BlockSpec((_RB, D), lambda i: (i, 0)),
        out_shape=jax.ShapeDtypeStruct((NPAD, D), jnp.float32),
    )(agg2, deg2, W_enc, b_enc, W1a, W1b, b1)


@functools.partial(
    pl.kernel,
    out_type=jax.ShapeDtypeStruct((EP,), jnp.float32),
    mesh=_mesh,
    scratch_types=[
        pltpu.VMEM((NB, KB), jnp.int32),
        pltpu.VMEM((NB, KB), jnp.int32),
        pltpu.VMEM((KB, D), jnp.float32),
        pltpu.VMEM((KB, D), jnp.float32),
        pltpu.VMEM((KB, D), jnp.float32),
        pltpu.VMEM((KB, D), jnp.float32),
        pltpu.VMEM((KB,), jnp.float32),
        pltpu.VMEM((H,), jnp.float32),
        pltpu.VMEM((L,), jnp.float32),
        pltpu.SemaphoreType.DMA,
        pltpu.SemaphoreType.DMA,
    ],
    compiler_params=_params,
)
def _edge_mlp(c_hbm, src_hbm, dst_hbm, w2_hbm, b2v_hbm, out_hbm,
              sidx_v, didx_v, cs0_v, cd0_v, cs1_v, cd1_v, out_v, w2_v, b2v_v,
              sem0, sem1):
    cid = lax.axis_index("c")
    sid = lax.axis_index("s")
    wid = sid * NC + cid

    pltpu.sync_copy(src_hbm.at[wid], sidx_v)
    pltpu.sync_copy(dst_hbm.at[wid], didx_v)
    pltpu.sync_copy(w2_hbm, w2_v)
    pltpu.sync_copy(b2v_hbm, b2v_v)
    w2 = [w2_v[pl.ds(j * L, L)] for j in range(H // L)]
    b2s = b2v_v[...]
    riota = jnp.arange(L, dtype=jnp.int32)

    def _issue(b, cs, cd, sem):
        _ = pltpu.async_copy(c_hbm.at[sidx_v.at[b]], cs, sem)
        _ = pltpu.async_copy(c_hbm.at[didx_v.at[b]], cd, sem)

    def _wait(b, cs, cd, sem):
        pltpu.make_async_copy(c_hbm.at[sidx_v.at[b]], cs, sem).wait()
        pltpu.make_async_copy(c_hbm.at[didx_v.at[b]], cd, sem).wait()

    lane0 = riota == 0

    def _compute(b, cs, cd):
        def _group(g, _):
            base = g * 8
            for i in range(8):
                row = base + i
                acc = b2s
                for j in range(H // L):
                    a = cs[row, pl.ds(j * L, L)]
                    bb = cd[row, pl.ds(H + j * L, L)]
                    acc = acc + jnp.maximum(a + bb, 0.0) * w2[j]
                r = jnp.sum(acc)
                plsc.store_scatter(out_v, [jnp.full((L,), row, jnp.int32)],
                                   jnp.full((L,), r, jnp.float32), mask=lane0)
            return ()

        lax.fori_loop(0, KB // 8, _group, ())
        pltpu.sync_copy(out_v, out_hbm.at[pl.ds(wid * EPW + b * KB, KB)])

    _issue(0, cs0_v, cd0_v, sem0)

    def _pair(p, _):
        b0 = 2 * p
        b1 = b0 + 1
        _wait(b0, cs0_v, cd0_v, sem0)
        _issue(b1, cs1_v, cd1_v, sem1)
        _compute(b0, cs0_v, cd0_v)
        _wait(b1, cs1_v, cd1_v, sem1)

        @pl.when(b0 + 2 < NB)
        def _():
            _issue(b0 + 2, cs0_v, cd0_v, sem0)

        _compute(b1, cs1_v, cd1_v)
        return ()

    lax.fori_loop(0, NPAIR, _pair, ())


def kernel(x, edge_index, W_enc, b_enc, W1, b1, W2, b2):
    src = edge_index[0]
    dst = edge_index[1]
    pad = EP - E
    ar = jnp.arange(pad, dtype=jnp.int32)
    srcp = jnp.concatenate([src, ar % N])
    dstp = jnp.concatenate([dst, N + ar % (NPAD - N)])
    src3 = srcp.reshape(NW, NB, KB)
    dst3 = dstp.reshape(NW, NB, KB)
    agg2, deg_flat = _seg_sum(x, src3, dst3)
    deg2 = deg_flat.reshape(NW, NPAD, 1)
    C = _node_stage(agg2, deg2, W_enc, b_enc, W1[:D], W1[D:], b1)
    b2v = jnp.full((L,), b2[0], jnp.float32)
    logits = _edge_mlp(C, src3, dst3, W2.reshape(H), b2v)
    return logits[:E].reshape(E, 1)


# speedup vs baseline: 3.7478x; 1.4505x over previous
import functools

import jax
import jax.numpy as jnp
from jax import lax
from jax.experimental import pallas as pl
from jax.experimental.pallas import tpu as pltpu
from jax.experimental.pallas import tpu_sc as plsc

N = 10000
E = 320000
D = 128
H = 64

NC = 2
NS = 16
NW = NC * NS
L = 16

NPAD = 10240
RPS = NPAD // NS

KB = 128
EP = 327680
EPW = EP // NW
NB = EPW // KB
NPAIR = NB // 2

_mesh = plsc.VectorSubcoreMesh(core_axis_name="c", subcore_axis_name="s")
_params = pltpu.CompilerParams(needs_layout_passes=False)


@functools.partial(
    pl.kernel,
    out_type=(jax.ShapeDtypeStruct((NC, NPAD, D), jnp.float32),
              jax.ShapeDtypeStruct((NW * NPAD,), jnp.float32)),
    mesh=_mesh,
    scratch_types=[
        pltpu.VMEM((NB, KB), jnp.int32),
        pltpu.VMEM((NB, KB), jnp.int32),
        pltpu.VMEM((KB, D), jnp.float32),
        pltpu.VMEM((KB, D), jnp.float32),
        pltpu.VMEM((NPAD,), jnp.float32),
        pltpu.VMEM_SHARED((NPAD, D), jnp.float32),
        pltpu.SemaphoreType.DMA,
        pltpu.SemaphoreType.DMA,
    ],
    compiler_params=_params,
)
def _seg_sum(x_hbm, src_hbm, dst_hbm, agg_hbm, deg_hbm,
             sidx_v, didx_v, rows0_v, rows1_v, hist_v, agg_sh, sem0, sem1):
    cid = lax.axis_index("c")
    sid = lax.axis_index("s")
    wid = sid * NC + cid

    zero = jnp.zeros((L,), jnp.float32)

    pltpu.sync_copy(src_hbm.at[wid], sidx_v)
    pltpu.sync_copy(dst_hbm.at[wid], didx_v)

    def _zrow(r, _):
        for j in range(D // L):
            rows0_v[r, pl.ds(j * L, L)] = zero
        return ()

    lax.fori_loop(0, KB, _zrow, ())

    def _zhist(g, _):
        hist_v[pl.ds(g * L, L)] = zero
        return ()

    lax.fori_loop(0, NPAD // L, _zhist, ())

    for c in range(RPS // KB):
        pltpu.sync_copy(rows0_v, agg_sh.at[pl.ds(sid * RPS + c * KB, KB), :])

    plsc.subcore_barrier()

    def _deg(b):
        for g in range(KB // L):
            dv = didx_v[b, pl.ds(g * L, L)]
            cnt, last = plsc.scan_count(dv)
            plsc.addupdate_scatter(hist_v, [dv], cnt.astype(jnp.float32),
                                   mask=last)

    def _batch(b, _):
        d = pltpu.async_copy(x_hbm.at[sidx_v.at[b]], rows0_v, sem0)
        _deg(b)
        d.wait()
        pltpu.sync_copy(rows0_v, agg_sh.at[didx_v.at[b]], add=True)
        return ()

    lax.fori_loop(0, NB, _batch, ())
    plsc.subcore_barrier()

    pltpu.sync_copy(agg_sh.at[pl.ds(sid * RPS, RPS), :],
                    agg_hbm.at[cid, pl.ds(sid * RPS, RPS), :])
    pltpu.sync_copy(hist_v, deg_hbm.at[pl.ds(wid * NPAD, NPAD)])


def _node_body(agg_ref, deg_ref, wenc_ref, benc_ref, w1a_ref, w1b_ref, b1_ref,
               c_ref):
    deg = jnp.maximum(jnp.sum(deg_ref[...], axis=1, keepdims=True), 1.0)
    s = agg_ref[0] + agg_ref[1]
    xb = s / deg
    ne = jnp.dot(xb, wenc_ref[...], preferred_element_type=jnp.float32)
    ne = jnp.maximum(ne + benc_ref[...][None, :], 0.0)
    a = jnp.dot(ne, w1a_ref[...], preferred_element_type=jnp.float32)
    b = (jnp.dot(ne, w1b_ref[...], preferred_element_type=jnp.float32)
         + b1_ref[...][None, :])
    c_ref[...] = jnp.concatenate([a, b], axis=1)


_RB = 1024


def _node_stage(agg2, deg2, W_enc, b_enc, W1a, W1b, b1):
    grid = NPAD // _RB
    return pl.pallas_call(
        _node_body,
        grid=(grid,),
        in_specs=[
            pl.BlockSpec((NC, _RB, D), lambda i: (0, i, 0)),
            pl.BlockSpec((_RB, NW), lambda i: (i, 0)),
            pl.BlockSpec((D, D), lambda i: (0, 0)),
            pl.BlockSpec((D,), lambda i: (0,)),
            pl.BlockSpec((D, H), lambda i: (0, 0)),
            pl.BlockSpec((D, H), lambda i: (0, 0)),
            pl.BlockSpec((H,), lambda i: (0,)),
        ],
        out_specs=pl.BlockSpec((_RB, D), lambda i: (i, 0)),
        out_shape=jax.ShapeDtypeStruct((NPAD, D), jnp.float32),
    )(agg2, deg2, W_enc, b_enc, W1a, W1b, b1)


@functools.partial(
    pl.kernel,
    out_type=jax.ShapeDtypeStruct((EP,), jnp.float32),
    mesh=_mesh,
    scratch_types=[
        pltpu.VMEM((NB, KB), jnp.int32),
        pltpu.VMEM((NB, KB), jnp.int32),
        pltpu.VMEM((KB, D), jnp.float32),
        pltpu.VMEM((KB, D), jnp.float32),
        pltpu.VMEM((KB, D), jnp.float32),
        pltpu.VMEM((KB, D), jnp.float32),
        pltpu.VMEM((KB,), jnp.float32),
        pltpu.VMEM((H,), jnp.float32),
        pltpu.VMEM((L,), jnp.float32),
        pltpu.SemaphoreType.DMA,
        pltpu.SemaphoreType.DMA,
    ],
    compiler_params=_params,
)
def _edge_mlp(c_hbm, src_hbm, dst_hbm, w2_hbm, b2v_hbm, out_hbm,
              sidx_v, didx_v, cs0_v, cd0_v, cs1_v, cd1_v, out_v, w2_v, b2v_v,
              sem0, sem1):
    cid = lax.axis_index("c")
    sid = lax.axis_index("s")
    wid = sid * NC + cid

    pltpu.sync_copy(src_hbm.at[wid], sidx_v)
    pltpu.sync_copy(dst_hbm.at[wid], didx_v)
    pltpu.sync_copy(w2_hbm, w2_v)
    pltpu.sync_copy(b2v_hbm, b2v_v)
    w2 = [w2_v[pl.ds(j * L, L)] for j in range(H // L)]
    b2s = b2v_v[...]
    riota = jnp.arange(L, dtype=jnp.int32)

    def _issue(b, cs, cd, sem):
        _ = pltpu.async_copy(c_hbm.at[sidx_v.at[b]], cs, sem)
        _ = pltpu.async_copy(c_hbm.at[didx_v.at[b]], cd, sem)

    def _wait(b, cs, cd, sem):
        pltpu.make_async_copy(c_hbm.at[sidx_v.at[b]], cs, sem).wait()
        pltpu.make_async_copy(c_hbm.at[didx_v.at[b]], cd, sem).wait()

    lane0 = riota == 0

    def _compute(b, cs, cd):
        def _group(g, _):
            base = g * 8
            for i in range(8):
                row = base + i
                acc = b2s
                for j in range(H // L):
                    a = cs[row, pl.ds(j * L, L)]
                    bb = cd[row, pl.ds(H + j * L, L)]
                    acc = acc + jnp.maximum(a + bb, 0.0) * w2[j]
                r = jnp.sum(acc)
                plsc.store_scatter(out_v, [jnp.full((L,), row, jnp.int32)],
                                   jnp.full((L,), r, jnp.float32), mask=lane0)
            return ()

        lax.fori_loop(0, KB // 8, _group, ())
        pltpu.sync_copy(out_v, out_hbm.at[pl.ds(wid * EPW + b * KB, KB)])

    _issue(0, cs0_v, cd0_v, sem0)

    def _pair(p, _):
        b0 = 2 * p
        b1 = b0 + 1
        _wait(b0, cs0_v, cd0_v, sem0)
        _issue(b1, cs1_v, cd1_v, sem1)
        _compute(b0, cs0_v, cd0_v)
        _wait(b1, cs1_v, cd1_v, sem1)

        @pl.when(b0 + 2 < NB)
        def _():
            _issue(b0 + 2, cs0_v, cd0_v, sem0)

        _compute(b1, cs1_v, cd1_v)
        return ()

    lax.fori_loop(0, NPAIR, _pair, ())


def kernel(x, edge_index, W_enc, b_enc, W1, b1, W2, b2):
    src = edge_index[0]
    dst = edge_index[1]
    pad = EP - E
    ar = jnp.arange(pad, dtype=jnp.int32)
    srcp = jnp.concatenate([src, ar % N])
    dstp = jnp.concatenate([dst, N + ar % (NPAD - N)])
    src3 = srcp.reshape(NW, NB, KB)
    dst3 = dstp.reshape(NW, NB, KB)
    agg2, deg_flat = _seg_sum(x, src3, dst3)
    deg2 = deg_flat.reshape(NW, NPAD).T
    C = _node_stage(agg2, deg2, W_enc, b_enc, W1[:D], W1[D:], b1)
    b2v = jnp.full((L,), b2[0], jnp.float32)
    logits = _edge_mlp(C, src3, dst3, W2.reshape(H), b2v)
    return logits[:E].reshape(E, 1)


# speedup vs baseline: 3.7596x; 1.0031x over previous
import functools

import jax
import jax.numpy as jnp
from jax import lax
from jax.experimental import pallas as pl
from jax.experimental.pallas import tpu as pltpu
from jax.experimental.pallas import tpu_sc as plsc

N = 10000
E = 320000
D = 128
H = 64

NC = 2
NS = 16
NW = NC * NS
L = 16

NPAD = 10240
RPS = NPAD // NS

KB = 128
EP = 327680
EPW = EP // NW
NB = EPW // KB
NPAIR = NB // 2

_mesh = plsc.VectorSubcoreMesh(core_axis_name="c", subcore_axis_name="s")
_params = pltpu.CompilerParams(needs_layout_passes=False)


@functools.partial(
    pl.kernel,
    out_type=(jax.ShapeDtypeStruct((NC, NPAD, D), jnp.float32),
              jax.ShapeDtypeStruct((NW * NPAD,), jnp.float32)),
    mesh=_mesh,
    scratch_types=[
        pltpu.VMEM((NB, KB), jnp.int32),
        pltpu.VMEM((NB, KB), jnp.int32),
        pltpu.VMEM((KB, D), jnp.float32),
        pltpu.VMEM((KB, D), jnp.float32),
        pltpu.VMEM((NPAD,), jnp.float32),
        pltpu.VMEM_SHARED((NPAD, D), jnp.float32),
        pltpu.SemaphoreType.DMA,
        pltpu.SemaphoreType.DMA,
    ],
    compiler_params=_params,
)
def _seg_sum(x_hbm, src_hbm, dst_hbm, agg_hbm, deg_hbm,
             sidx_v, didx_v, rows0_v, rows1_v, hist_v, agg_sh, sem0, sem1):
    cid = lax.axis_index("c")
    sid = lax.axis_index("s")
    wid = sid * NC + cid

    zero = jnp.zeros((L,), jnp.float32)

    pltpu.sync_copy(src_hbm.at[wid], sidx_v)
    pltpu.sync_copy(dst_hbm.at[wid], didx_v)

    def _zrow(r, _):
        for j in range(D // L):
            rows0_v[r, pl.ds(j * L, L)] = zero
        return ()

    lax.fori_loop(0, KB, _zrow, ())

    def _zhist(g, _):
        hist_v[pl.ds(g * L, L)] = zero
        return ()

    lax.fori_loop(0, NPAD // L, _zhist, ())

    for c in range(RPS // KB):
        pltpu.sync_copy(rows0_v, agg_sh.at[pl.ds(sid * RPS + c * KB, KB), :])

    plsc.subcore_barrier()

    def _deg(b):
        for g in range(KB // L):
            dv = didx_v[b, pl.ds(g * L, L)]
            cnt, last = plsc.scan_count(dv)
            plsc.addupdate_scatter(hist_v, [dv], cnt.astype(jnp.float32),
                                   mask=last)

    def _batch(b, _):
        d = pltpu.async_copy(x_hbm.at[sidx_v.at[b]], rows0_v, sem0)
        _deg(b)
        d.wait()
        pltpu.sync_copy(rows0_v, agg_sh.at[didx_v.at[b]], add=True)
        return ()

    lax.fori_loop(0, NB, _batch, ())
    plsc.subcore_barrier()

    pltpu.sync_copy(agg_sh.at[pl.ds(sid * RPS, RPS), :],
                    agg_hbm.at[cid, pl.ds(sid * RPS, RPS), :])
    pltpu.sync_copy(hist_v, deg_hbm.at[pl.ds(wid * NPAD, NPAD)])


def _node_body(agg_ref, deg_ref, wenc_ref, benc_ref, w1a_ref, w1b_ref, b1_ref,
               c_ref):
    deg = jnp.maximum(jnp.sum(deg_ref[...], axis=1, keepdims=True), 1.0)
    s = agg_ref[0] + agg_ref[1]
    xb = s / deg
    ne = jnp.dot(xb, wenc_ref[...], preferred_element_type=jnp.float32)
    ne = jnp.maximum(ne + benc_ref[...][None, :], 0.0)
    a = jnp.dot(ne, w1a_ref[...], preferred_element_type=jnp.float32)
    b = (jnp.dot(ne, w1b_ref[...], preferred_element_type=jnp.float32)
         + b1_ref[...][None, :])
    c_ref[...] = jnp.concatenate([a, b], axis=1)


_RB = 1024


def _node_stage(agg2, deg2, W_enc, b_enc, W1a, W1b, b1):
    grid = NPAD // _RB
    return pl.pallas_call(
        _node_body,
        grid=(grid,),
        in_specs=[
            pl.BlockSpec((NC, _RB, D), lambda i: (0, i, 0)),
            pl.BlockSpec((_RB, NW), lambda i: (i, 0)),
            pl.BlockSpec((D, D), lambda i: (0, 0)),
            pl.BlockSpec((D,), lambda i: (0,)),
            pl.BlockSpec((D, H), lambda i: (0, 0)),
            pl.BlockSpec((D, H), lambda i: (0, 0)),
            pl.BlockSpec((H,), lambda i: (0,)),
        ],
        out_specs=pl.BlockSpec((_RB, D), lambda i: (i, 0)),
        out_shape=jax.ShapeDtypeStruct((NPAD, D), jnp.float32),
    )(agg2, deg2, W_enc, b_enc, W1a, W1b, b1)


@functools.partial(
    pl.kernel,
    out_type=jax.ShapeDtypeStruct((E,), jnp.float32),
    mesh=_mesh,
    scratch_types=[
        pltpu.VMEM((NB, KB), jnp.int32),
        pltpu.VMEM((NB, KB), jnp.int32),
        pltpu.VMEM((KB, D), jnp.float32),
        pltpu.VMEM((KB, D), jnp.float32),
        pltpu.VMEM((KB, D), jnp.float32),
        pltpu.VMEM((KB, D), jnp.float32),
        pltpu.VMEM((KB,), jnp.float32),
        pltpu.VMEM((H,), jnp.float32),
        pltpu.VMEM((L,), jnp.float32),
        pltpu.SemaphoreType.DMA,
        pltpu.SemaphoreType.DMA,
    ],
    compiler_params=_params,
)
def _edge_mlp(c_hbm, src_hbm, dst_hbm, w2_hbm, b2v_hbm, out_hbm,
              sidx_v, didx_v, cs0_v, cd0_v, cs1_v, cd1_v, out_v, w2_v, b2v_v,
              sem0, sem1):
    cid = lax.axis_index("c")
    sid = lax.axis_index("s")
    wid = sid * NC + cid

    pltpu.sync_copy(src_hbm.at[wid], sidx_v)
    pltpu.sync_copy(dst_hbm.at[wid], didx_v)
    pltpu.sync_copy(w2_hbm, w2_v)
    pltpu.sync_copy(b2v_hbm, b2v_v)
    w2 = [w2_v[pl.ds(j * L, L)] for j in range(H // L)]
    b2s = b2v_v[...]
    riota = jnp.arange(L, dtype=jnp.int32)

    def _issue(b, cs, cd, sem):
        _ = pltpu.async_copy(c_hbm.at[sidx_v.at[b]], cs, sem)
        _ = pltpu.async_copy(c_hbm.at[didx_v.at[b]], cd, sem)

    def _wait(b, cs, cd, sem):
        pltpu.make_async_copy(c_hbm.at[sidx_v.at[b]], cs, sem).wait()
        pltpu.make_async_copy(c_hbm.at[didx_v.at[b]], cd, sem).wait()

    lane0 = riota == 0

    def _compute(b, cs, cd):
        def _group(g, _):
            base = g * 8
            for i in range(8):
                row = base + i
                acc = b2s
                for j in range(H // L):
                    a = cs[row, pl.ds(j * L, L)]
                    bb = cd[row, pl.ds(H + j * L, L)]
                    acc = acc + jnp.maximum(a + bb, 0.0) * w2[j]
                r = jnp.sum(acc)
                plsc.store_scatter(out_v, [jnp.full((L,), row, jnp.int32)],
                                   jnp.full((L,), r, jnp.float32), mask=lane0)
            return ()

        lax.fori_loop(0, KB // 8, _group, ())

        @pl.when(wid * EPW + b * KB < E)
        def _():
            pltpu.sync_copy(out_v, out_hbm.at[pl.ds(wid * EPW + b * KB, KB)])

    _issue(0, cs0_v, cd0_v, sem0)

    def _pair(p, _):
        b0 = 2 * p
        b1 = b0 + 1
        _wait(b0, cs0_v, cd0_v, sem0)
        _issue(b1, cs1_v, cd1_v, sem1)
        _compute(b0, cs0_v, cd0_v)
        _wait(b1, cs1_v, cd1_v, sem1)

        @pl.when(b0 + 2 < NB)
        def _():
            _issue(b0 + 2, cs0_v, cd0_v, sem0)

        _compute(b1, cs1_v, cd1_v)
        return ()

    lax.fori_loop(0, NPAIR, _pair, ())


def kernel(x, edge_index, W_enc, b_enc, W1, b1, W2, b2):
    src = edge_index[0]
    dst = edge_index[1]
    pad = EP - E
    ar = jnp.arange(pad, dtype=jnp.int32)
    srcp = jnp.concatenate([src, ar % N])
    dstp = jnp.concatenate([dst, N + ar % (NPAD - N)])
    src3 = srcp.reshape(NW, NB, KB)
    dst3 = dstp.reshape(NW, NB, KB)
    agg2, deg_flat = _seg_sum(x, src3, dst3)
    deg2 = deg_flat.reshape(NW, NPAD).T
    C = _node_stage(agg2, deg2, W_enc, b_enc, W1[:D], W1[D:], b1)
    b2v = jnp.full((L,), b2[0], jnp.float32)
    logits = _edge_mlp(C, src3, dst3, W2.reshape(H), b2v)
    return logits.reshape(E, 1)


# speedup vs baseline: 3.7625x; 1.0008x over previous
import functools

import jax
import jax.numpy as jnp
from jax import lax
from jax.experimental import pallas as pl
from jax.experimental.pallas import tpu as pltpu
from jax.experimental.pallas import tpu_sc as plsc

N = 10000
E = 320000
D = 128
H = 64

NC = 2
NS = 16
NW = NC * NS
L = 16

NPAD = 10240
RPS = NPAD // NS

KB = 128
KB1 = 64
NB1 = 10240 // KB1
NP1 = NB1 // 2
EP = 327680
EPW = EP // NW
NB = EPW // KB
NPAIR = NB // 2

_mesh = plsc.VectorSubcoreMesh(core_axis_name="c", subcore_axis_name="s")
_params = pltpu.CompilerParams(needs_layout_passes=False)


@functools.partial(
    pl.kernel,
    out_type=(jax.ShapeDtypeStruct((NC, NPAD, D), jnp.float32),
              jax.ShapeDtypeStruct((NW * NPAD,), jnp.float32)),
    mesh=_mesh,
    scratch_types=[
        pltpu.VMEM((NB, KB), jnp.int32),
        pltpu.VMEM((NB, KB), jnp.int32),
        pltpu.VMEM((KB, D), jnp.float32),
        pltpu.VMEM((KB, D), jnp.float32),
        pltpu.VMEM((NPAD,), jnp.float32),
        pltpu.VMEM_SHARED((NPAD, D), jnp.float32),
        pltpu.SemaphoreType.DMA,
        pltpu.SemaphoreType.DMA,
    ],
    compiler_params=_params,
)
def _seg_sum(x_hbm, src_hbm, dst_hbm, agg_hbm, deg_hbm,
             sidx_v, didx_v, rows0_v, rows1_v, hist_v, agg_sh, sem0, sem1):
    cid = lax.axis_index("c")
    sid = lax.axis_index("s")
    wid = sid * NC + cid

    zero = jnp.zeros((L,), jnp.float32)

    pltpu.sync_copy(src_hbm.at[wid], sidx_v)
    pltpu.sync_copy(dst_hbm.at[wid], didx_v)

    def _zrow(r, _):
        for j in range(D // L):
            rows0_v[r, pl.ds(j * L, L)] = zero
        return ()

    lax.fori_loop(0, KB, _zrow, ())

    def _zhist(g, _):
        hist_v[pl.ds(g * L, L)] = zero
        return ()

    lax.fori_loop(0, NPAD // L, _zhist, ())

    for c in range(RPS // KB):
        pltpu.sync_copy(rows0_v, agg_sh.at[pl.ds(sid * RPS + c * KB, KB), :])

    plsc.subcore_barrier()

    def _deg(b):
        for g in range(KB // L):
            dv = didx_v[b, pl.ds(g * L, L)]
            cnt, last = plsc.scan_count(dv)
            plsc.addupdate_scatter(hist_v, [dv], cnt.astype(jnp.float32),
                                   mask=last)

    def _batch(b, _):
        d = pltpu.async_copy(x_hbm.at[sidx_v.at[b]], rows0_v, sem0)
        _deg(b)
        d.wait()
        pltpu.sync_copy(rows0_v, agg_sh.at[didx_v.at[b]], add=True)
        return ()

    lax.fori_loop(0, NB, _batch, ())
    plsc.subcore_barrier()

    pltpu.sync_copy(agg_sh.at[pl.ds(sid * RPS, RPS), :],
                    agg_hbm.at[cid, pl.ds(sid * RPS, RPS), :])
    pltpu.sync_copy(hist_v, deg_hbm.at[pl.ds(wid * NPAD, NPAD)])


def _node_body(agg_ref, deg_ref, wenc_ref, benc_ref, w1a_ref, w1b_ref, b1_ref,
               c_ref):
    deg = jnp.maximum(jnp.sum(deg_ref[...], axis=1, keepdims=True), 1.0)
    s = agg_ref[0] + agg_ref[1]
    xb = s / deg
    ne = jnp.dot(xb, wenc_ref[...], preferred_element_type=jnp.float32)
    ne = jnp.maximum(ne + benc_ref[...][None, :], 0.0)
    a = jnp.dot(ne, w1a_ref[...], preferred_element_type=jnp.float32)
    b = (jnp.dot(ne, w1b_ref[...], preferred_element_type=jnp.float32)
         + b1_ref[...][None, :])
    c_ref[...] = jnp.concatenate([a, b], axis=1)


_RB = 1024


def _node_stage(agg2, deg2, W_enc, b_enc, W1a, W1b, b1):
    grid = NPAD // _RB
    return pl.pallas_call(
        _node_body,
        grid=(grid,),
        in_specs=[
            pl.BlockSpec((NC, _RB, D), lambda i: (0, i, 0)),
            pl.BlockSpec((_RB, NW), lambda i: (i, 0)),
            pl.BlockSpec((D, D), lambda i: (0, 0)),
            pl.BlockSpec((D,), lambda i: (0,)),
            pl.BlockSpec((D, H), lambda i: (0, 0)),
            pl.BlockSpec((D, H), lambda i: (0, 0)),
            pl.BlockSpec((H,), lambda i: (0,)),
        ],
        out_specs=pl.BlockSpec((_RB, D), lambda i: (i, 0)),
        out_shape=jax.ShapeDtypeStruct((NPAD, D), jnp.float32),
    )(agg2, deg2, W_enc, b_enc, W1a, W1b, b1)


@functools.partial(
    pl.kernel,
    out_type=jax.ShapeDtypeStruct((E,), jnp.float32),
    mesh=_mesh,
    scratch_types=[
        pltpu.VMEM((NB, KB), jnp.int32),
        pltpu.VMEM((NB, KB), jnp.int32),
        pltpu.VMEM((KB, D), jnp.float32),
        pltpu.VMEM((KB, D), jnp.float32),
        pltpu.VMEM((KB, D), jnp.float32),
        pltpu.VMEM((KB, D), jnp.float32),
        pltpu.VMEM((KB,), jnp.float32),
        pltpu.VMEM((H,), jnp.float32),
        pltpu.VMEM((L,), jnp.float32),
        pltpu.SemaphoreType.DMA,
        pltpu.SemaphoreType.DMA,
    ],
    compiler_params=_params,
)
def _edge_mlp(c_hbm, src_hbm, dst_hbm, w2_hbm, b2v_hbm, out_hbm,
              sidx_v, didx_v, cs0_v, cd0_v, cs1_v, cd1_v, out_v, w2_v, b2v_v,
              sem0, sem1):
    cid = lax.axis_index("c")
    sid = lax.axis_index("s")
    wid = sid * NC + cid

    pltpu.sync_copy(src_hbm.at[wid], sidx_v)
    pltpu.sync_copy(dst_hbm.at[wid], didx_v)
    pltpu.sync_copy(w2_hbm, w2_v)
    pltpu.sync_copy(b2v_hbm, b2v_v)
    w2 = [w2_v[pl.ds(j * L, L)] for j in range(H // L)]
    b2s = b2v_v[...]
    riota = jnp.arange(L, dtype=jnp.int32)

    def _issue(b, cs, cd, sem):
        _ = pltpu.async_copy(c_hbm.at[sidx_v.at[b]], cs, sem)
        _ = pltpu.async_copy(c_hbm.at[didx_v.at[b]], cd, sem)

    def _wait(b, cs, cd, sem):
        pltpu.make_async_copy(c_hbm.at[sidx_v.at[b]], cs, sem).wait()
        pltpu.make_async_copy(c_hbm.at[didx_v.at[b]], cd, sem).wait()

    lane0 = riota == 0

    def _compute(b, cs, cd):
        def _group(g, _):
            base = g * 8
            for i in range(8):
                row = base + i
                acc = b2s
                for j in range(H // L):
                    a = cs[row, pl.ds(j * L, L)]
                    bb = cd[row, pl.ds(H + j * L, L)]
                    acc = acc + jnp.maximum(a + bb, 0.0) * w2[j]
                r = jnp.sum(acc)
                plsc.store_scatter(out_v, [jnp.full((L,), row, jnp.int32)],
                                   jnp.full((L,), r, jnp.float32), mask=lane0)
            return ()

        lax.fori_loop(0, KB // 8, _group, ())

        @pl.when(wid * EPW + b * KB < E)
        def _():
            pltpu.sync_copy(out_v, out_hbm.at[pl.ds(wid * EPW + b * KB, KB)])

    _issue(0, cs0_v, cd0_v, sem0)

    def _pair(p, _):
        b0 = 2 * p
        b1 = b0 + 1
        _wait(b0, cs0_v, cd0_v, sem0)
        _issue(b1, cs1_v, cd1_v, sem1)
        _compute(b0, cs0_v, cd0_v)
        _wait(b1, cs1_v, cd1_v, sem1)

        @pl.when(b0 + 2 < NB)
        def _():
            _issue(b0 + 2, cs0_v, cd0_v, sem0)

        _compute(b1, cs1_v, cd1_v)
        return ()

    lax.fori_loop(0, NPAIR, _pair, ())


def kernel(x, edge_index, W_enc, b_enc, W1, b1, W2, b2):
    src = edge_index[0]
    dst = edge_index[1]
    pad = EP - E
    ar = jnp.arange(pad, dtype=jnp.int32)
    srcp = jnp.concatenate([src, ar % N])
    dstp = jnp.concatenate([dst, N + ar % (NPAD - N)])
    src3 = srcp.reshape(NW, NB, KB)
    dst3 = dstp.reshape(NW, NB, KB)
    agg2, deg_flat = _seg_sum(x, src3, dst3)
    deg2 = deg_flat.reshape(NW, NPAD).T
    C = _node_stage(agg2, deg2, W_enc, b_enc, W1[:D], W1[D:], b1)
    b2v = jnp.full((L,), b2[0], jnp.float32)
    logits = _edge_mlp(C, src3, dst3, W2.reshape(H), b2v)
    return logits.reshape(E, 1)
